# baseline (speedup 1.0000x reference)
import functools

import jax
import jax.numpy as jnp
from jax import lax
from jax.experimental import pallas as pl
from jax.experimental.pallas import tpu as pltpu
from jax.experimental.pallas import tpu_sc as plsc

N_NODES = 10000
N_EDGES = 320000
D_FEAT = 128

NC = 2
NS = 16
NW = NC * NS
L = 16

EPW = N_EDGES // NW
NPAD = 10240
CH = NPAD // NS
NPASS = 3
SEG = 3584
RPT = SEG // NS
K = 112
LSZ = EPW + K + L
NBMAX = (EPW + K - 1) // K

_f32 = jnp.float32
_i32 = jnp.int32


def _zero_vmem(ref, n):
    def body(i, _):
        ref[pl.ds(i * L, L)] = jnp.zeros((L,), _f32)
        return 0
    lax.fori_loop(0, n // L, body, 0)


def _newton_rsqrt(d):
    i = plsc.bitcast(d, _i32)
    i = jnp.full((L,), 0x5F3759DF, _i32) - lax.shift_right_arithmetic(
        i, jnp.full((L,), 1, _i32))
    y = plsc.bitcast(i, _f32)
    half = d * 0.5
    for _ in range(3):
        y = y * (1.5 - half * y * y)
    return y


_SC_PARAMS = dict(
    compiler_params=pltpu.CompilerParams(
        needs_layout_passes=False, use_tc_tiling_on_sc=False),
)


def _make_deg_kernel():
    mesh = plsc.VectorSubcoreMesh(core_axis_name="c", subcore_axis_name="s")

    @functools.partial(
        pl.kernel,
        mesh=mesh,
        out_type=jax.ShapeDtypeStruct((NC, NPAD), _f32),
        scratch_types=[
            pltpu.VMEM((EPW,), _i32),
            pltpu.VMEM((EPW,), _f32),
            pltpu.VMEM((NPAD,), _f32),
            pltpu.VMEM_SHARED((NS, NPAD), _f32),
            pltpu.VMEM((CH,), _f32),
            pltpu.VMEM((CH,), _f32),
        ],
        **_SC_PARAMS,
    )
    def deg_kernel(dst_hbm, w_hbm, out_hbm, dst_v, w_v, deg_v, shared, tmp_v,
                   acc_v):
        c = lax.axis_index("c")
        s = lax.axis_index("s")
        base = (c * NS + s) * EPW
        pltpu.sync_copy(dst_hbm.at[pl.ds(base, EPW)], dst_v)
        pltpu.sync_copy(w_hbm.at[pl.ds(base, EPW)], w_v)
        _zero_vmem(deg_v, NPAD)

        def scatter_body(i, _):
            idx = dst_v[pl.ds(i * L, L)]
            wv = w_v[pl.ds(i * L, L)]
            plsc.addupdate_scatter(deg_v, [idx], wv)
            return 0
        lax.fori_loop(0, EPW // L, scatter_body, 0)

        pltpu.sync_copy(deg_v, shared.at[s])
        plsc.subcore_barrier()

        _zero_vmem(acc_v, CH)

        def red_body(t, _):
            pltpu.sync_copy(shared.at[t, pl.ds(s * CH, CH)], tmp_v)

            def add_body(i, _):
                acc_v[pl.ds(i * L, L)] = (
                    acc_v[pl.ds(i * L, L)] + tmp_v[pl.ds(i * L, L)])
                return 0
            lax.fori_loop(0, CH // L, add_body, 0)
            return 0
        lax.fori_loop(0, NS, red_body, 0)

        pltpu.sync_copy(acc_v, out_hbm.at[c, pl.ds(s * CH, CH)])

    return deg_kernel


def _make_main_kernel():
    mesh = plsc.VectorSubcoreMesh(core_axis_name="c", subcore_axis_name="s")

    @functools.partial(
        pl.kernel,
        mesh=mesh,
        out_type=(
            jax.ShapeDtypeStruct((N_NODES, D_FEAT), _f32),
            jax.ShapeDtypeStruct((N_NODES, D_FEAT), _f32),
        ),
        scratch_types=[
            pltpu.VMEM((NPAD,), _f32),
            pltpu.VMEM((EPW,), _i32),
            pltpu.VMEM((EPW,), _i32),
            pltpu.VMEM((EPW,), _f32),
            pltpu.VMEM((LSZ,), _i32),
            pltpu.VMEM((LSZ,), _i32),
            pltpu.VMEM((LSZ,), _f32),
            pltpu.VMEM((K,), _i32),
            pltpu.VMEM((K,), _i32),
            pltpu.VMEM((K,), _i32),
            pltpu.VMEM((K,), _i32),
            pltpu.VMEM((K, D_FEAT), _f32),
            pltpu.VMEM((K, D_FEAT), _f32),
            pltpu.VMEM((CH,), _f32),
            pltpu.VMEM((CH,), _f32),
            pltpu.VMEM_SHARED((NPAD,), _f32),
            pltpu.VMEM_SHARED((SEG, D_FEAT), _f32),
            pltpu.SemaphoreType.DMA,
            pltpu.SemaphoreType.DMA,
        ],
        **_SC_PARAMS,
    )
    def main_kernel(x_hbm, src_hbm, dst_hbm, w_hbm, degp_hbm, out0_hbm,
                    out1_hbm, dis_v, src_v, dst_v, w_v, sl_v, dl_v, nl_v,
                    gi0_v, si0_v, gi1_v, si1_v, rows0_v, rows1_v, t0_v, t1_v,
                    shared_dis, acc, sem0, sem1):
        c = lax.axis_index("c")
        s = lax.axis_index("s")
        base = (c * NS + s) * EPW

        pltpu.sync_copy(degp_hbm.at[0, pl.ds(s * CH, CH)], t0_v)
        pltpu.sync_copy(degp_hbm.at[1, pl.ds(s * CH, CH)], t1_v)

        def dis_body(i, _):
            d = t0_v[pl.ds(i * L, L)] + t1_v[pl.ds(i * L, L)]
            y = _newton_rsqrt(jnp.maximum(d, 1e-12))
            t0_v[pl.ds(i * L, L)] = jnp.where(d > 0.0, y, 0.0)
            return 0
        lax.fori_loop(0, CH // L, dis_body, 0)
        pltpu.sync_copy(t0_v, shared_dis.at[pl.ds(s * CH, CH)])
        plsc.subcore_barrier()
        pltpu.sync_copy(shared_dis, dis_v)

        pltpu.sync_copy(src_hbm.at[pl.ds(base, EPW)], src_v)
        pltpu.sync_copy(dst_hbm.at[pl.ds(base, EPW)], dst_v)
        pltpu.sync_copy(w_hbm.at[pl.ds(base, EPW)], w_v)

        def norm_body(i, _):
            sv = src_v[pl.ds(i * L, L)]
            dv = dst_v[pl.ds(i * L, L)]
            w_v[pl.ds(i * L, L)] = (
                plsc.load_gather(dis_v, [sv]) * w_v[pl.ds(i * L, L)]
                * plsc.load_gather(dis_v, [dv]))
            return 0
        lax.fori_loop(0, EPW // L, norm_body, 0)

        def fill_idx(gi, si, b):
            for g in range(K // L):
                gi[pl.ds(g * L, L)] = sl_v[pl.ds(b * K + g * L, L)]
                si[pl.ds(g * L, L)] = dl_v[pl.ds(b * K + g * L, L)]

        def scale(rows, b):
            def scale_g(g, _):
                for jj in range(L):
                    j = g * L + jj
                    nsp = plsc.load_gather(
                        nl_v, [jnp.full((L,), b * K, _i32) + j])
                    for cb in range(D_FEAT // L):
                        rows[j, pl.ds(cb * L, L)] = (
                            rows[j, pl.ds(cb * L, L)] * nsp)
                return 0
            lax.fori_loop(0, K // L, scale_g, 0)

        for p in range(NPASS):
            def zrows_body(j, _):
                for cb in range(D_FEAT // L):
                    rows0_v[j, pl.ds(cb * L, L)] = jnp.zeros((L,), _f32)
                return 0
            lax.fori_loop(0, K, zrows_body, 0)
            for r in range(RPT // K):
                pltpu.sync_copy(rows0_v, acc.at[pl.ds(s * RPT + r * K, K)])
            ated = (RPT // K) * K
            if ated < RPT:
                pltpu.sync_copy(rows0_v.at[pl.ds(0, RPT - ated)],
                                acc.at[pl.ds(s * RPT + ated, RPT - ated)])
            plsc.subcore_barrier()

            def cmp_body(i, cnt):
                dv = dst_v[pl.ds(i * L, L)] - (p * SEG)
                msk = (dv >= 0) & (dv < SEG)
                plsc.store_compressed(sl_v.at[pl.ds(cnt, L)],
                                      src_v[pl.ds(i * L, L)], mask=msk)
                plsc.store_compressed(dl_v.at[pl.ds(cnt, L)], dv, mask=msk)
                plsc.store_compressed(nl_v.at[pl.ds(cnt, L)],
                                      w_v[pl.ds(i * L, L)], mask=msk)
                npop = jnp.max(plsc.all_reduce_population_count(msk))
                return cnt + npop
            cnt = lax.fori_loop(0, EPW // L, cmp_body, 0)

            for t in range(K // L):
                nl_v[pl.ds(cnt + t * L, L)] = jnp.zeros((L,), _f32)
                sl_v[pl.ds(cnt + t * L, L)] = jnp.zeros((L,), _i32)
                dl_v[pl.ds(cnt + t * L, L)] = jnp.zeros((L,), _i32)
            nb = (cnt + (K - 1)) // K

            @pl.when(nb > 0)
            def _():
                fill_idx(gi0_v, si0_v, 0)
                pltpu.async_copy(x_hbm.at[gi0_v], rows0_v, sem0)

            def pair_body(i, _):
                b0 = 2 * i
                b1 = b0 + 1

                @pl.when(b1 < nb)
                def _():
                    fill_idx(gi1_v, si1_v, b1)
                    pltpu.async_copy(x_hbm.at[gi1_v], rows1_v, sem1)

                pltpu.make_async_copy(x_hbm.at[gi0_v], rows0_v, sem0).wait()
                scale(rows0_v, b0)
                pltpu.sync_copy(rows0_v, acc.at[si0_v], add=True)

                @pl.when(b1 < nb)
                def _():
                    @pl.when(b0 + 2 < nb)
                    def _():
                        fill_idx(gi0_v, si0_v, b0 + 2)
                        pltpu.async_copy(x_hbm.at[gi0_v], rows0_v, sem0)

                    pltpu.make_async_copy(
                        x_hbm.at[gi1_v], rows1_v, sem1).wait()
                    scale(rows1_v, b1)
                    pltpu.sync_copy(rows1_v, acc.at[si1_v], add=True)
                return 0
            lax.fori_loop(0, (nb + 1) // 2, pair_body, 0)
            plsc.subcore_barrier()

            row0 = p * SEG + s * RPT
            lim = min(SEG, N_NODES - p * SEG)

            def wb(dst_ref):
                if lim == SEG:
                    pltpu.sync_copy(acc.at[pl.ds(s * RPT, RPT)],
                                    dst_ref.at[pl.ds(row0, RPT)])
                else:
                    @pl.when(s * RPT + RPT <= lim)
                    def _():
                        pltpu.sync_copy(acc.at[pl.ds(s * RPT, RPT)],
                                        dst_ref.at[pl.ds(row0, RPT)])

                    @pl.when((s * RPT < lim) & (s * RPT + RPT > lim))
                    def _():
                        part = lim % RPT
                        pltpu.sync_copy(acc.at[pl.ds(s * RPT, part)],
                                        dst_ref.at[pl.ds(row0, part)])

            @pl.when(c == 0)
            def _():
                wb(out0_hbm)

            @pl.when(c == 1)
            def _():
                wb(out1_hbm)

            plsc.subcore_barrier()

    return main_kernel


def _add_tc(a_ref, b_ref, o_ref):
    o_ref[...] = a_ref[...] + b_ref[...]


def kernel(x, edge_index, edge_weight):
    src = edge_index[0].astype(_i32)
    dst = edge_index[1].astype(_i32)
    w = edge_weight.astype(_f32)

    degp = _make_deg_kernel()(dst, w)
    p0, p1 = _make_main_kernel()(x, src, dst, w, degp)

    blk = 400
    out = pl.pallas_call(
        _add_tc,
        grid=(N_NODES // blk,),
        in_specs=[pl.BlockSpec((blk, D_FEAT), lambda i: (i, 0))] * 2,
        out_specs=pl.BlockSpec((blk, D_FEAT), lambda i: (i, 0)),
        out_shape=jax.ShapeDtypeStruct((N_NODES, D_FEAT), _f32),
    )(p0, p1)
    return out


# speedup vs baseline: 1.1488x; 1.1488x over previous
import functools

import jax
import jax.numpy as jnp
from jax import lax
from jax.experimental import pallas as pl
from jax.experimental.pallas import tpu as pltpu
from jax.experimental.pallas import tpu_sc as plsc

N_NODES = 10000
N_EDGES = 320000
D_FEAT = 128

NC = 2
NS = 16
NW = NC * NS
L = 16

EPW = N_EDGES // NW
NPAD = 10240
CH = NPAD // NS
NPASS = 3
SEG = 3584
RPT = SEG // NS
K = 80
LSZ = EPW + K + L
NBMAX = (EPW + K - 1) // K

_f32 = jnp.float32
_i32 = jnp.int32


def _zero_vmem(ref, n):
    def body(i, _):
        ref[pl.ds(i * L, L)] = jnp.zeros((L,), _f32)
        return 0
    lax.fori_loop(0, n // L, body, 0)


def _newton_rsqrt(d):
    i = plsc.bitcast(d, _i32)
    i = jnp.full((L,), 0x5F3759DF, _i32) - lax.shift_right_arithmetic(
        i, jnp.full((L,), 1, _i32))
    y = plsc.bitcast(i, _f32)
    half = d * 0.5
    for _ in range(3):
        y = y * (1.5 - half * y * y)
    return y


_SC_PARAMS = dict(
    compiler_params=pltpu.CompilerParams(
        needs_layout_passes=False, use_tc_tiling_on_sc=False),
)


def _make_deg_kernel():
    mesh = plsc.VectorSubcoreMesh(core_axis_name="c", subcore_axis_name="s")

    @functools.partial(
        pl.kernel,
        mesh=mesh,
        out_type=jax.ShapeDtypeStruct((NC, NPAD), _f32),
        scratch_types=[
            pltpu.VMEM((EPW,), _i32),
            pltpu.VMEM((EPW,), _f32),
            pltpu.VMEM((NPAD,), _f32),
            pltpu.VMEM_SHARED((NS, NPAD), _f32),
            pltpu.VMEM((CH,), _f32),
            pltpu.VMEM((CH,), _f32),
        ],
        **_SC_PARAMS,
    )
    def deg_kernel(dst_hbm, w_hbm, out_hbm, dst_v, w_v, deg_v, shared, tmp_v,
                   acc_v):
        c = lax.axis_index("c")
        s = lax.axis_index("s")
        base = (c * NS + s) * EPW
        pltpu.sync_copy(dst_hbm.at[pl.ds(base, EPW)], dst_v)
        pltpu.sync_copy(w_hbm.at[pl.ds(base, EPW)], w_v)
        _zero_vmem(deg_v, NPAD)

        def scatter_body(i, _):
            idx = dst_v[pl.ds(i * L, L)]
            wv = w_v[pl.ds(i * L, L)]
            plsc.addupdate_scatter(deg_v, [idx], wv)
            return 0
        lax.fori_loop(0, EPW // L, scatter_body, 0)

        pltpu.sync_copy(deg_v, shared.at[s])
        plsc.subcore_barrier()

        _zero_vmem(acc_v, CH)

        def red_body(t, _):
            pltpu.sync_copy(shared.at[t, pl.ds(s * CH, CH)], tmp_v)

            def add_body(i, _):
                acc_v[pl.ds(i * L, L)] = (
                    acc_v[pl.ds(i * L, L)] + tmp_v[pl.ds(i * L, L)])
                return 0
            lax.fori_loop(0, CH // L, add_body, 0)
            return 0
        lax.fori_loop(0, NS, red_body, 0)

        pltpu.sync_copy(acc_v, out_hbm.at[c, pl.ds(s * CH, CH)])

    return deg_kernel


def _make_main_kernel():
    mesh = plsc.VectorSubcoreMesh(core_axis_name="c", subcore_axis_name="s")

    @functools.partial(
        pl.kernel,
        mesh=mesh,
        out_type=(
            jax.ShapeDtypeStruct((N_NODES, D_FEAT), _f32),
            jax.ShapeDtypeStruct((N_NODES, D_FEAT), _f32),
        ),
        scratch_types=[
            pltpu.VMEM((NPAD,), _f32),
            pltpu.VMEM((EPW,), _i32),
            pltpu.VMEM((EPW,), _i32),
            pltpu.VMEM((EPW,), _f32),
            pltpu.VMEM((LSZ,), _i32),
            pltpu.VMEM((LSZ,), _i32),
            pltpu.VMEM((LSZ,), _f32),
            pltpu.VMEM((K,), _i32),
            pltpu.VMEM((K,), _i32),
            pltpu.VMEM((K,), _i32),
            pltpu.VMEM((K,), _i32),
            pltpu.VMEM((K, D_FEAT), _f32),
            pltpu.VMEM((K, D_FEAT), _f32),
            pltpu.VMEM((CH,), _f32),
            pltpu.VMEM((CH,), _f32),
            pltpu.VMEM_SHARED((NPAD,), _f32),
            pltpu.VMEM_SHARED((SEG, D_FEAT), _f32),
            pltpu.SemaphoreType.DMA,
            pltpu.SemaphoreType.DMA,
        ],
        **_SC_PARAMS,
    )
    def main_kernel(x_hbm, src_hbm, dst_hbm, w_hbm, degp_hbm, out0_hbm,
                    out1_hbm, dis_v, src_v, dst_v, w_v, sl_v, dl_v, nl_v,
                    gi0_v, si0_v, gi1_v, si1_v, rows0_v, rows1_v, t0_v, t1_v,
                    shared_dis, acc, sem0, sem1):
        c = lax.axis_index("c")
        s = lax.axis_index("s")
        base = (c * NS + s) * EPW

        pltpu.sync_copy(degp_hbm.at[0, pl.ds(s * CH, CH)], t0_v)
        pltpu.sync_copy(degp_hbm.at[1, pl.ds(s * CH, CH)], t1_v)

        def dis_body(i, _):
            d = t0_v[pl.ds(i * L, L)] + t1_v[pl.ds(i * L, L)]
            y = _newton_rsqrt(jnp.maximum(d, 1e-12))
            t0_v[pl.ds(i * L, L)] = jnp.where(d > 0.0, y, 0.0)
            return 0
        lax.fori_loop(0, CH // L, dis_body, 0)
        pltpu.sync_copy(t0_v, shared_dis.at[pl.ds(s * CH, CH)])
        plsc.subcore_barrier()
        pltpu.sync_copy(shared_dis, dis_v)

        pltpu.sync_copy(src_hbm.at[pl.ds(base, EPW)], src_v)
        pltpu.sync_copy(dst_hbm.at[pl.ds(base, EPW)], dst_v)
        pltpu.sync_copy(w_hbm.at[pl.ds(base, EPW)], w_v)

        def norm_body(i, _):
            sv = src_v[pl.ds(i * L, L)]
            dv = dst_v[pl.ds(i * L, L)]
            w_v[pl.ds(i * L, L)] = (
                plsc.load_gather(dis_v, [sv]) * w_v[pl.ds(i * L, L)]
                * plsc.load_gather(dis_v, [dv]))
            return 0
        lax.fori_loop(0, EPW // L, norm_body, 0)

        def fill_idx(gi, si, b):
            for g in range(K // L):
                gi[pl.ds(g * L, L)] = sl_v[pl.ds(b * K + g * L, L)]
                si[pl.ds(g * L, L)] = dl_v[pl.ds(b * K + g * L, L)]

        def scale(rows, b):
            def scale_g(g, _):
                for jj in range(L):
                    j = g * L + jj
                    nsp = plsc.load_gather(
                        nl_v, [jnp.full((L,), b * K, _i32) + j])
                    for cb in range(D_FEAT // L):
                        rows[j, pl.ds(cb * L, L)] = (
                            rows[j, pl.ds(cb * L, L)] * nsp)
                return 0
            lax.fori_loop(0, K // L, scale_g, 0)

        for p in range(NPASS):
            def zrows_body(j, _):
                for cb in range(D_FEAT // L):
                    rows0_v[j, pl.ds(cb * L, L)] = jnp.zeros((L,), _f32)
                return 0
            lax.fori_loop(0, K, zrows_body, 0)
            for r in range(RPT // K):
                pltpu.sync_copy(rows0_v, acc.at[pl.ds(s * RPT + r * K, K)])
            ated = (RPT // K) * K
            if ated < RPT:
                pltpu.sync_copy(rows0_v.at[pl.ds(0, RPT - ated)],
                                acc.at[pl.ds(s * RPT + ated, RPT - ated)])
            plsc.subcore_barrier()

            def cmp_body(i, cnt):
                dv = dst_v[pl.ds(i * L, L)] - (p * SEG)
                msk = (dv >= 0) & (dv < SEG)
                plsc.store_compressed(sl_v.at[pl.ds(cnt, L)],
                                      src_v[pl.ds(i * L, L)], mask=msk)
                plsc.store_compressed(dl_v.at[pl.ds(cnt, L)], dv, mask=msk)
                plsc.store_compressed(nl_v.at[pl.ds(cnt, L)],
                                      w_v[pl.ds(i * L, L)], mask=msk)
                npop = jnp.max(plsc.all_reduce_population_count(msk))
                return cnt + npop
            cnt = lax.fori_loop(0, EPW // L, cmp_body, 0)

            for t in range(K // L):
                nl_v[pl.ds(cnt + t * L, L)] = jnp.zeros((L,), _f32)
                sl_v[pl.ds(cnt + t * L, L)] = jnp.zeros((L,), _i32)
                dl_v[pl.ds(cnt + t * L, L)] = jnp.zeros((L,), _i32)
            nb = (cnt + (K - 1)) // K

            @pl.when(nb > 0)
            def _():
                fill_idx(gi0_v, si0_v, 0)
                pltpu.async_copy(x_hbm.at[gi0_v], rows0_v, sem0)

            def pair_body(i, _):
                b0 = 2 * i
                b1 = b0 + 1

                @pl.when(b1 < nb)
                def _():
                    fill_idx(gi1_v, si1_v, b1)
                    pltpu.async_copy(x_hbm.at[gi1_v], rows1_v, sem1)

                pltpu.make_async_copy(x_hbm.at[gi0_v], rows0_v, sem0).wait()
                scale(rows0_v, b0)
                pltpu.sync_copy(rows0_v, acc.at[si0_v], add=True)

                @pl.when(b1 < nb)
                def _():
                    @pl.when(b0 + 2 < nb)
                    def _():
                        fill_idx(gi0_v, si0_v, b0 + 2)
                        pltpu.async_copy(x_hbm.at[gi0_v], rows0_v, sem0)

                    pltpu.make_async_copy(
                        x_hbm.at[gi1_v], rows1_v, sem1).wait()
                    scale(rows1_v, b1)
                    pltpu.sync_copy(rows1_v, acc.at[si1_v], add=True)
                return 0
            lax.fori_loop(0, (nb + 1) // 2, pair_body, 0)
            plsc.subcore_barrier()

            row0 = p * SEG + s * RPT
            lim = min(SEG, N_NODES - p * SEG)

            def wb(dst_ref):
                if lim == SEG:
                    pltpu.sync_copy(acc.at[pl.ds(s * RPT, RPT)],
                                    dst_ref.at[pl.ds(row0, RPT)])
                else:
                    @pl.when(s * RPT + RPT <= lim)
                    def _():
                        pltpu.sync_copy(acc.at[pl.ds(s * RPT, RPT)],
                                        dst_ref.at[pl.ds(row0, RPT)])

                    @pl.when((s * RPT < lim) & (s * RPT + RPT > lim))
                    def _():
                        part = lim % RPT
                        pltpu.sync_copy(acc.at[pl.ds(s * RPT, part)],
                                        dst_ref.at[pl.ds(row0, part)])

            @pl.when(c == 0)
            def _():
                wb(out0_hbm)

            @pl.when(c == 1)
            def _():
                wb(out1_hbm)

            plsc.subcore_barrier()

    return main_kernel


def _add_tc(a_ref, b_ref, o_ref):
    o_ref[...] = a_ref[...] + b_ref[...]


def kernel(x, edge_index, edge_weight):
    src = edge_index[0].astype(_i32)
    dst = edge_index[1].astype(_i32)
    w = edge_weight.astype(_f32)

    degp = _make_deg_kernel()(dst, w)
    p0, p1 = _make_main_kernel()(x, src, dst, w, degp)

    blk = 400
    out = pl.pallas_call(
        _add_tc,
        grid=(N_NODES // blk,),
        in_specs=[pl.BlockSpec((blk, D_FEAT), lambda i: (i, 0))] * 2,
        out_specs=pl.BlockSpec((blk, D_FEAT), lambda i: (i, 0)),
        out_shape=jax.ShapeDtypeStruct((N_NODES, D_FEAT), _f32),
    )(p0, p1)
    return out


# speedup vs baseline: 1.2366x; 1.0764x over previous
import functools

import jax
import jax.numpy as jnp
from jax import lax
from jax.experimental import pallas as pl
from jax.experimental.pallas import tpu as pltpu
from jax.experimental.pallas import tpu_sc as plsc

N_NODES = 10000
N_EDGES = 320000
D_FEAT = 128

NC = 2
NS = 16
NW = NC * NS
L = 16

EPW = N_EDGES // NW
NPAD = 10240
CH = NPAD // NS
NPASS = 3
SEG = 3584
RPT = SEG // NS
K = 64
LSZ = EPW + K + L
NBMAX = (EPW + K - 1) // K

_f32 = jnp.float32
_i32 = jnp.int32


def _zero_vmem(ref, n):
    def body(i, _):
        ref[pl.ds(i * L, L)] = jnp.zeros((L,), _f32)
        return 0
    lax.fori_loop(0, n // L, body, 0)


def _newton_rsqrt(d):
    i = plsc.bitcast(d, _i32)
    i = jnp.full((L,), 0x5F3759DF, _i32) - lax.shift_right_arithmetic(
        i, jnp.full((L,), 1, _i32))
    y = plsc.bitcast(i, _f32)
    half = d * 0.5
    for _ in range(3):
        y = y * (1.5 - half * y * y)
    return y


_SC_PARAMS = dict(
    compiler_params=pltpu.CompilerParams(
        needs_layout_passes=False, use_tc_tiling_on_sc=False),
)


def _make_deg_kernel():
    mesh = plsc.VectorSubcoreMesh(core_axis_name="c", subcore_axis_name="s")

    @functools.partial(
        pl.kernel,
        mesh=mesh,
        out_type=jax.ShapeDtypeStruct((NC, NPAD), _f32),
        scratch_types=[
            pltpu.VMEM((EPW,), _i32),
            pltpu.VMEM((EPW,), _f32),
            pltpu.VMEM((NPAD,), _f32),
            pltpu.VMEM_SHARED((NS, NPAD), _f32),
            pltpu.VMEM((CH,), _f32),
            pltpu.VMEM((CH,), _f32),
        ],
        **_SC_PARAMS,
    )
    def deg_kernel(dst_hbm, w_hbm, out_hbm, dst_v, w_v, deg_v, shared, tmp_v,
                   acc_v):
        c = lax.axis_index("c")
        s = lax.axis_index("s")
        base = (c * NS + s) * EPW
        pltpu.sync_copy(dst_hbm.at[pl.ds(base, EPW)], dst_v)
        pltpu.sync_copy(w_hbm.at[pl.ds(base, EPW)], w_v)
        _zero_vmem(deg_v, NPAD)

        def scatter_body(i, _):
            idx = dst_v[pl.ds(i * L, L)]
            wv = w_v[pl.ds(i * L, L)]
            plsc.addupdate_scatter(deg_v, [idx], wv)
            return 0
        lax.fori_loop(0, EPW // L, scatter_body, 0)

        pltpu.sync_copy(deg_v, shared.at[s])
        plsc.subcore_barrier()

        _zero_vmem(acc_v, CH)

        def red_body(t, _):
            pltpu.sync_copy(shared.at[t, pl.ds(s * CH, CH)], tmp_v)

            def add_body(i, _):
                acc_v[pl.ds(i * L, L)] = (
                    acc_v[pl.ds(i * L, L)] + tmp_v[pl.ds(i * L, L)])
                return 0
            lax.fori_loop(0, CH // L, add_body, 0)
            return 0
        lax.fori_loop(0, NS, red_body, 0)

        pltpu.sync_copy(acc_v, out_hbm.at[c, pl.ds(s * CH, CH)])

    return deg_kernel


def _make_main_kernel():
    mesh = plsc.VectorSubcoreMesh(core_axis_name="c", subcore_axis_name="s")

    @functools.partial(
        pl.kernel,
        mesh=mesh,
        out_type=(
            jax.ShapeDtypeStruct((N_NODES, D_FEAT), _f32),
            jax.ShapeDtypeStruct((N_NODES, D_FEAT), _f32),
        ),
        scratch_types=[
            pltpu.VMEM((NPAD,), _f32),
            pltpu.VMEM((EPW,), _i32),
            pltpu.VMEM((EPW,), _i32),
            pltpu.VMEM((EPW,), _f32),
            pltpu.VMEM((LSZ,), _i32),
            pltpu.VMEM((LSZ,), _i32),
            pltpu.VMEM((LSZ,), _f32),
            pltpu.VMEM((K,), _i32),
            pltpu.VMEM((K,), _i32),
            pltpu.VMEM((K,), _i32),
            pltpu.VMEM((K,), _i32),
            pltpu.VMEM((K, D_FEAT), _f32),
            pltpu.VMEM((K, D_FEAT), _f32),
            pltpu.VMEM((CH,), _f32),
            pltpu.VMEM((CH,), _f32),
            pltpu.VMEM_SHARED((NPAD,), _f32),
            pltpu.VMEM_SHARED((SEG, D_FEAT), _f32),
            pltpu.SemaphoreType.DMA,
            pltpu.SemaphoreType.DMA,
        ],
        **_SC_PARAMS,
    )
    def main_kernel(x_hbm, src_hbm, dst_hbm, w_hbm, degp_hbm, out0_hbm,
                    out1_hbm, dis_v, src_v, dst_v, w_v, sl_v, dl_v, nl_v,
                    gi0_v, si0_v, gi1_v, si1_v, rows0_v, rows1_v, t0_v, t1_v,
                    shared_dis, acc, sem0, sem1):
        c = lax.axis_index("c")
        s = lax.axis_index("s")
        base = (c * NS + s) * EPW

        pltpu.sync_copy(degp_hbm.at[0, pl.ds(s * CH, CH)], t0_v)
        pltpu.sync_copy(degp_hbm.at[1, pl.ds(s * CH, CH)], t1_v)

        def dis_body(i, _):
            d = t0_v[pl.ds(i * L, L)] + t1_v[pl.ds(i * L, L)]
            y = _newton_rsqrt(jnp.maximum(d, 1e-12))
            t0_v[pl.ds(i * L, L)] = jnp.where(d > 0.0, y, 0.0)
            return 0
        lax.fori_loop(0, CH // L, dis_body, 0)
        pltpu.sync_copy(t0_v, shared_dis.at[pl.ds(s * CH, CH)])
        plsc.subcore_barrier()
        pltpu.sync_copy(shared_dis, dis_v)

        pltpu.sync_copy(src_hbm.at[pl.ds(base, EPW)], src_v)
        pltpu.sync_copy(dst_hbm.at[pl.ds(base, EPW)], dst_v)
        pltpu.sync_copy(w_hbm.at[pl.ds(base, EPW)], w_v)

        def norm_body(i, _):
            sv = src_v[pl.ds(i * L, L)]
            dv = dst_v[pl.ds(i * L, L)]
            w_v[pl.ds(i * L, L)] = (
                plsc.load_gather(dis_v, [sv]) * w_v[pl.ds(i * L, L)]
                * plsc.load_gather(dis_v, [dv]))
            return 0
        lax.fori_loop(0, EPW // L, norm_body, 0)

        def fill_idx(gi, si, b):
            for g in range(K // L):
                gi[pl.ds(g * L, L)] = sl_v[pl.ds(b * K + g * L, L)]
                si[pl.ds(g * L, L)] = dl_v[pl.ds(b * K + g * L, L)]

        def scale(rows, b):
            def scale_g(g, _):
                for jj in range(L):
                    j = g * L + jj
                    nsp = plsc.load_gather(
                        nl_v, [jnp.full((L,), b * K, _i32) + j])
                    for cb in range(D_FEAT // L):
                        rows[j, pl.ds(cb * L, L)] = (
                            rows[j, pl.ds(cb * L, L)] * nsp)
                return 0
            lax.fori_loop(0, K // L, scale_g, 0)

        for p in range(NPASS):
            def zrows_body(j, _):
                for cb in range(D_FEAT // L):
                    rows0_v[j, pl.ds(cb * L, L)] = jnp.zeros((L,), _f32)
                return 0
            lax.fori_loop(0, K, zrows_body, 0)
            for r in range(RPT // K):
                pltpu.sync_copy(rows0_v, acc.at[pl.ds(s * RPT + r * K, K)])
            ated = (RPT // K) * K
            if ated < RPT:
                pltpu.sync_copy(rows0_v.at[pl.ds(0, RPT - ated)],
                                acc.at[pl.ds(s * RPT + ated, RPT - ated)])
            plsc.subcore_barrier()

            def cmp_body(i, cnt):
                dv = dst_v[pl.ds(i * L, L)] - (p * SEG)
                msk = (dv >= 0) & (dv < SEG)
                plsc.store_compressed(sl_v.at[pl.ds(cnt, L)],
                                      src_v[pl.ds(i * L, L)], mask=msk)
                plsc.store_compressed(dl_v.at[pl.ds(cnt, L)], dv, mask=msk)
                plsc.store_compressed(nl_v.at[pl.ds(cnt, L)],
                                      w_v[pl.ds(i * L, L)], mask=msk)
                npop = jnp.max(plsc.all_reduce_population_count(msk))
                return cnt + npop
            cnt = lax.fori_loop(0, EPW // L, cmp_body, 0)

            for t in range(K // L):
                nl_v[pl.ds(cnt + t * L, L)] = jnp.zeros((L,), _f32)
                sl_v[pl.ds(cnt + t * L, L)] = jnp.zeros((L,), _i32)
                dl_v[pl.ds(cnt + t * L, L)] = jnp.zeros((L,), _i32)
            nb = (cnt + (K - 1)) // K

            @pl.when(nb > 0)
            def _():
                fill_idx(gi0_v, si0_v, 0)
                pltpu.async_copy(x_hbm.at[gi0_v], rows0_v, sem0)

            def pair_body(i, _):
                b0 = 2 * i
                b1 = b0 + 1

                @pl.when(b1 < nb)
                def _():
                    fill_idx(gi1_v, si1_v, b1)
                    pltpu.async_copy(x_hbm.at[gi1_v], rows1_v, sem1)

                pltpu.make_async_copy(x_hbm.at[gi0_v], rows0_v, sem0).wait()
                scale(rows0_v, b0)
                pltpu.sync_copy(rows0_v, acc.at[si0_v], add=True)

                @pl.when(b1 < nb)
                def _():
                    @pl.when(b0 + 2 < nb)
                    def _():
                        fill_idx(gi0_v, si0_v, b0 + 2)
                        pltpu.async_copy(x_hbm.at[gi0_v], rows0_v, sem0)

                    pltpu.make_async_copy(
                        x_hbm.at[gi1_v], rows1_v, sem1).wait()
                    scale(rows1_v, b1)
                    pltpu.sync_copy(rows1_v, acc.at[si1_v], add=True)
                return 0
            lax.fori_loop(0, (nb + 1) // 2, pair_body, 0)
            plsc.subcore_barrier()

            row0 = p * SEG + s * RPT
            lim = min(SEG, N_NODES - p * SEG)

            def wb(dst_ref):
                if lim == SEG:
                    pltpu.sync_copy(acc.at[pl.ds(s * RPT, RPT)],
                                    dst_ref.at[pl.ds(row0, RPT)])
                else:
                    @pl.when(s * RPT + RPT <= lim)
                    def _():
                        pltpu.sync_copy(acc.at[pl.ds(s * RPT, RPT)],
                                        dst_ref.at[pl.ds(row0, RPT)])

                    @pl.when((s * RPT < lim) & (s * RPT + RPT > lim))
                    def _():
                        part = lim % RPT
                        pltpu.sync_copy(acc.at[pl.ds(s * RPT, part)],
                                        dst_ref.at[pl.ds(row0, part)])

            @pl.when(c == 0)
            def _():
                wb(out0_hbm)

            @pl.when(c == 1)
            def _():
                wb(out1_hbm)

            plsc.subcore_barrier()

    return main_kernel


def _add_tc(a_ref, b_ref, o_ref):
    o_ref[...] = a_ref[...] + b_ref[...]


def kernel(x, edge_index, edge_weight):
    src = edge_index[0].astype(_i32)
    dst = edge_index[1].astype(_i32)
    w = edge_weight.astype(_f32)

    degp = _make_deg_kernel()(dst, w)
    p0, p1 = _make_main_kernel()(x, src, dst, w, degp)

    blk = 400
    out = pl.pallas_call(
        _add_tc,
        grid=(N_NODES // blk,),
        in_specs=[pl.BlockSpec((blk, D_FEAT), lambda i: (i, 0))] * 2,
        out_specs=pl.BlockSpec((blk, D_FEAT), lambda i: (i, 0)),
        out_shape=jax.ShapeDtypeStruct((N_NODES, D_FEAT), _f32),
    )(p0, p1)
    return out


# speedup vs baseline: 1.2555x; 1.0153x over previous
import functools

import jax
import jax.numpy as jnp
from jax import lax
from jax.experimental import pallas as pl
from jax.experimental.pallas import tpu as pltpu
from jax.experimental.pallas import tpu_sc as plsc

N_NODES = 10000
N_EDGES = 320000
D_FEAT = 128

NC = 2
NS = 16
NW = NC * NS
L = 16

EPW = N_EDGES // NW
NPAD = 10240
CH = NPAD // NS
NPASS = 3
SEG = 3584
RPT = SEG // NS
K = 48
LSZ = EPW + K + L
NBMAX = (EPW + K - 1) // K

_f32 = jnp.float32
_i32 = jnp.int32


def _zero_vmem(ref, n):
    def body(i, _):
        ref[pl.ds(i * L, L)] = jnp.zeros((L,), _f32)
        return 0
    lax.fori_loop(0, n // L, body, 0)


def _newton_rsqrt(d):
    i = plsc.bitcast(d, _i32)
    i = jnp.full((L,), 0x5F3759DF, _i32) - lax.shift_right_arithmetic(
        i, jnp.full((L,), 1, _i32))
    y = plsc.bitcast(i, _f32)
    half = d * 0.5
    for _ in range(3):
        y = y * (1.5 - half * y * y)
    return y


_SC_PARAMS = dict(
    compiler_params=pltpu.CompilerParams(
        needs_layout_passes=False, use_tc_tiling_on_sc=False),
)


def _make_deg_kernel():
    mesh = plsc.VectorSubcoreMesh(core_axis_name="c", subcore_axis_name="s")

    @functools.partial(
        pl.kernel,
        mesh=mesh,
        out_type=jax.ShapeDtypeStruct((NC, NPAD), _f32),
        scratch_types=[
            pltpu.VMEM((EPW,), _i32),
            pltpu.VMEM((EPW,), _f32),
            pltpu.VMEM((NPAD,), _f32),
            pltpu.VMEM_SHARED((NS, NPAD), _f32),
            pltpu.VMEM((CH,), _f32),
            pltpu.VMEM((CH,), _f32),
        ],
        **_SC_PARAMS,
    )
    def deg_kernel(dst_hbm, w_hbm, out_hbm, dst_v, w_v, deg_v, shared, tmp_v,
                   acc_v):
        c = lax.axis_index("c")
        s = lax.axis_index("s")
        base = (c * NS + s) * EPW
        pltpu.sync_copy(dst_hbm.at[pl.ds(base, EPW)], dst_v)
        pltpu.sync_copy(w_hbm.at[pl.ds(base, EPW)], w_v)
        _zero_vmem(deg_v, NPAD)

        def scatter_body(i, _):
            idx = dst_v[pl.ds(i * L, L)]
            wv = w_v[pl.ds(i * L, L)]
            plsc.addupdate_scatter(deg_v, [idx], wv)
            return 0
        lax.fori_loop(0, EPW // L, scatter_body, 0)

        pltpu.sync_copy(deg_v, shared.at[s])
        plsc.subcore_barrier()

        _zero_vmem(acc_v, CH)

        def red_body(t, _):
            pltpu.sync_copy(shared.at[t, pl.ds(s * CH, CH)], tmp_v)

            def add_body(i, _):
                acc_v[pl.ds(i * L, L)] = (
                    acc_v[pl.ds(i * L, L)] + tmp_v[pl.ds(i * L, L)])
                return 0
            lax.fori_loop(0, CH // L, add_body, 0)
            return 0
        lax.fori_loop(0, NS, red_body, 0)

        pltpu.sync_copy(acc_v, out_hbm.at[c, pl.ds(s * CH, CH)])

    return deg_kernel


def _make_main_kernel():
    mesh = plsc.VectorSubcoreMesh(core_axis_name="c", subcore_axis_name="s")

    @functools.partial(
        pl.kernel,
        mesh=mesh,
        out_type=(
            jax.ShapeDtypeStruct((N_NODES, D_FEAT), _f32),
            jax.ShapeDtypeStruct((N_NODES, D_FEAT), _f32),
        ),
        scratch_types=[
            pltpu.VMEM((NPAD,), _f32),
            pltpu.VMEM((EPW,), _i32),
            pltpu.VMEM((EPW,), _i32),
            pltpu.VMEM((EPW,), _f32),
            pltpu.VMEM((LSZ,), _i32),
            pltpu.VMEM((LSZ,), _i32),
            pltpu.VMEM((LSZ,), _f32),
            pltpu.VMEM((K,), _i32),
            pltpu.VMEM((K,), _i32),
            pltpu.VMEM((K,), _i32),
            pltpu.VMEM((K,), _i32),
            pltpu.VMEM((K, D_FEAT), _f32),
            pltpu.VMEM((K, D_FEAT), _f32),
            pltpu.VMEM((CH,), _f32),
            pltpu.VMEM((CH,), _f32),
            pltpu.VMEM_SHARED((NPAD,), _f32),
            pltpu.VMEM_SHARED((SEG, D_FEAT), _f32),
            pltpu.SemaphoreType.DMA,
            pltpu.SemaphoreType.DMA,
        ],
        **_SC_PARAMS,
    )
    def main_kernel(x_hbm, src_hbm, dst_hbm, w_hbm, degp_hbm, out0_hbm,
                    out1_hbm, dis_v, src_v, dst_v, w_v, sl_v, dl_v, nl_v,
                    gi0_v, si0_v, gi1_v, si1_v, rows0_v, rows1_v, t0_v, t1_v,
                    shared_dis, acc, sem0, sem1):
        c = lax.axis_index("c")
        s = lax.axis_index("s")
        base = (c * NS + s) * EPW

        pltpu.sync_copy(degp_hbm.at[0, pl.ds(s * CH, CH)], t0_v)
        pltpu.sync_copy(degp_hbm.at[1, pl.ds(s * CH, CH)], t1_v)

        def dis_body(i, _):
            d = t0_v[pl.ds(i * L, L)] + t1_v[pl.ds(i * L, L)]
            y = _newton_rsqrt(jnp.maximum(d, 1e-12))
            t0_v[pl.ds(i * L, L)] = jnp.where(d > 0.0, y, 0.0)
            return 0
        lax.fori_loop(0, CH // L, dis_body, 0)
        pltpu.sync_copy(t0_v, shared_dis.at[pl.ds(s * CH, CH)])
        plsc.subcore_barrier()
        pltpu.sync_copy(shared_dis, dis_v)

        pltpu.sync_copy(src_hbm.at[pl.ds(base, EPW)], src_v)
        pltpu.sync_copy(dst_hbm.at[pl.ds(base, EPW)], dst_v)
        pltpu.sync_copy(w_hbm.at[pl.ds(base, EPW)], w_v)

        def norm_body(i, _):
            sv = src_v[pl.ds(i * L, L)]
            dv = dst_v[pl.ds(i * L, L)]
            w_v[pl.ds(i * L, L)] = (
                plsc.load_gather(dis_v, [sv]) * w_v[pl.ds(i * L, L)]
                * plsc.load_gather(dis_v, [dv]))
            return 0
        lax.fori_loop(0, EPW // L, norm_body, 0)

        def fill_idx(gi, si, b):
            for g in range(K // L):
                gi[pl.ds(g * L, L)] = sl_v[pl.ds(b * K + g * L, L)]
                si[pl.ds(g * L, L)] = dl_v[pl.ds(b * K + g * L, L)]

        def scale(rows, b):
            def scale_g(g, _):
                for jj in range(L):
                    j = g * L + jj
                    nsp = plsc.load_gather(
                        nl_v, [jnp.full((L,), b * K, _i32) + j])
                    for cb in range(D_FEAT // L):
                        rows[j, pl.ds(cb * L, L)] = (
                            rows[j, pl.ds(cb * L, L)] * nsp)
                return 0
            lax.fori_loop(0, K // L, scale_g, 0)

        for p in range(NPASS):
            def zrows_body(j, _):
                for cb in range(D_FEAT // L):
                    rows0_v[j, pl.ds(cb * L, L)] = jnp.zeros((L,), _f32)
                return 0
            lax.fori_loop(0, K, zrows_body, 0)
            for r in range(RPT // K):
                pltpu.sync_copy(rows0_v, acc.at[pl.ds(s * RPT + r * K, K)])
            ated = (RPT // K) * K
            if ated < RPT:
                pltpu.sync_copy(rows0_v.at[pl.ds(0, RPT - ated)],
                                acc.at[pl.ds(s * RPT + ated, RPT - ated)])
            plsc.subcore_barrier()

            def cmp_body(i, cnt):
                dv = dst_v[pl.ds(i * L, L)] - (p * SEG)
                msk = (dv >= 0) & (dv < SEG)
                plsc.store_compressed(sl_v.at[pl.ds(cnt, L)],
                                      src_v[pl.ds(i * L, L)], mask=msk)
                plsc.store_compressed(dl_v.at[pl.ds(cnt, L)], dv, mask=msk)
                plsc.store_compressed(nl_v.at[pl.ds(cnt, L)],
                                      w_v[pl.ds(i * L, L)], mask=msk)
                npop = jnp.max(plsc.all_reduce_population_count(msk))
                return cnt + npop
            cnt = lax.fori_loop(0, EPW // L, cmp_body, 0)

            for t in range(K // L):
                nl_v[pl.ds(cnt + t * L, L)] = jnp.zeros((L,), _f32)
                sl_v[pl.ds(cnt + t * L, L)] = jnp.zeros((L,), _i32)
                dl_v[pl.ds(cnt + t * L, L)] = jnp.zeros((L,), _i32)
            nb = (cnt + (K - 1)) // K

            @pl.when(nb > 0)
            def _():
                fill_idx(gi0_v, si0_v, 0)
                pltpu.async_copy(x_hbm.at[gi0_v], rows0_v, sem0)

            def pair_body(i, _):
                b0 = 2 * i
                b1 = b0 + 1

                @pl.when(b1 < nb)
                def _():
                    fill_idx(gi1_v, si1_v, b1)
                    pltpu.async_copy(x_hbm.at[gi1_v], rows1_v, sem1)

                pltpu.make_async_copy(x_hbm.at[gi0_v], rows0_v, sem0).wait()
                scale(rows0_v, b0)
                pltpu.sync_copy(rows0_v, acc.at[si0_v], add=True)

                @pl.when(b1 < nb)
                def _():
                    @pl.when(b0 + 2 < nb)
                    def _():
                        fill_idx(gi0_v, si0_v, b0 + 2)
                        pltpu.async_copy(x_hbm.at[gi0_v], rows0_v, sem0)

                    pltpu.make_async_copy(
                        x_hbm.at[gi1_v], rows1_v, sem1).wait()
                    scale(rows1_v, b1)
                    pltpu.sync_copy(rows1_v, acc.at[si1_v], add=True)
                return 0
            lax.fori_loop(0, (nb + 1) // 2, pair_body, 0)
            plsc.subcore_barrier()

            row0 = p * SEG + s * RPT
            lim = min(SEG, N_NODES - p * SEG)

            def wb(dst_ref):
                if lim == SEG:
                    pltpu.sync_copy(acc.at[pl.ds(s * RPT, RPT)],
                                    dst_ref.at[pl.ds(row0, RPT)])
                else:
                    @pl.when(s * RPT + RPT <= lim)
                    def _():
                        pltpu.sync_copy(acc.at[pl.ds(s * RPT, RPT)],
                                        dst_ref.at[pl.ds(row0, RPT)])

                    @pl.when((s * RPT < lim) & (s * RPT + RPT > lim))
                    def _():
                        part = lim % RPT
                        pltpu.sync_copy(acc.at[pl.ds(s * RPT, part)],
                                        dst_ref.at[pl.ds(row0, part)])

            @pl.when(c == 0)
            def _():
                wb(out0_hbm)

            @pl.when(c == 1)
            def _():
                wb(out1_hbm)

            plsc.subcore_barrier()

    return main_kernel


def _add_tc(a_ref, b_ref, o_ref):
    o_ref[...] = a_ref[...] + b_ref[...]


def kernel(x, edge_index, edge_weight):
    src = edge_index[0].astype(_i32)
    dst = edge_index[1].astype(_i32)
    w = edge_weight.astype(_f32)

    degp = _make_deg_kernel()(dst, w)
    p0, p1 = _make_main_kernel()(x, src, dst, w, degp)

    blk = 400
    out = pl.pallas_call(
        _add_tc,
        grid=(N_NODES // blk,),
        in_specs=[pl.BlockSpec((blk, D_FEAT), lambda i: (i, 0))] * 2,
        out_specs=pl.BlockSpec((blk, D_FEAT), lambda i: (i, 0)),
        out_shape=jax.ShapeDtypeStruct((N_NODES, D_FEAT), _f32),
    )(p0, p1)
    return out


# speedup vs baseline: 1.2608x; 1.0042x over previous
import functools

import jax
import jax.numpy as jnp
from jax import lax
from jax.experimental import pallas as pl
from jax.experimental.pallas import tpu as pltpu
from jax.experimental.pallas import tpu_sc as plsc

N_NODES = 10000
N_EDGES = 320000
D_FEAT = 128

NC = 2
NS = 16
NW = NC * NS
L = 16

EPW = N_EDGES // NW
NPAD = 10240
CH = NPAD // NS
NPASS = 3
SEG = 3584
RPT = SEG // NS
K = 32
LSZ = EPW + K + L
NBMAX = (EPW + K - 1) // K

_f32 = jnp.float32
_i32 = jnp.int32


def _zero_vmem(ref, n):
    def body(i, _):
        ref[pl.ds(i * L, L)] = jnp.zeros((L,), _f32)
        return 0
    lax.fori_loop(0, n // L, body, 0)


def _newton_rsqrt(d):
    i = plsc.bitcast(d, _i32)
    i = jnp.full((L,), 0x5F3759DF, _i32) - lax.shift_right_arithmetic(
        i, jnp.full((L,), 1, _i32))
    y = plsc.bitcast(i, _f32)
    half = d * 0.5
    for _ in range(3):
        y = y * (1.5 - half * y * y)
    return y


_SC_PARAMS = dict(
    compiler_params=pltpu.CompilerParams(
        needs_layout_passes=False, use_tc_tiling_on_sc=False),
)


def _make_deg_kernel():
    mesh = plsc.VectorSubcoreMesh(core_axis_name="c", subcore_axis_name="s")

    @functools.partial(
        pl.kernel,
        mesh=mesh,
        out_type=jax.ShapeDtypeStruct((NC, NPAD), _f32),
        scratch_types=[
            pltpu.VMEM((EPW,), _i32),
            pltpu.VMEM((EPW,), _f32),
            pltpu.VMEM((NPAD,), _f32),
            pltpu.VMEM_SHARED((NS, NPAD), _f32),
            pltpu.VMEM((CH,), _f32),
            pltpu.VMEM((CH,), _f32),
        ],
        **_SC_PARAMS,
    )
    def deg_kernel(dst_hbm, w_hbm, out_hbm, dst_v, w_v, deg_v, shared, tmp_v,
                   acc_v):
        c = lax.axis_index("c")
        s = lax.axis_index("s")
        base = (c * NS + s) * EPW
        pltpu.sync_copy(dst_hbm.at[pl.ds(base, EPW)], dst_v)
        pltpu.sync_copy(w_hbm.at[pl.ds(base, EPW)], w_v)
        _zero_vmem(deg_v, NPAD)

        def scatter_body(i, _):
            idx = dst_v[pl.ds(i * L, L)]
            wv = w_v[pl.ds(i * L, L)]
            plsc.addupdate_scatter(deg_v, [idx], wv)
            return 0
        lax.fori_loop(0, EPW // L, scatter_body, 0)

        pltpu.sync_copy(deg_v, shared.at[s])
        plsc.subcore_barrier()

        _zero_vmem(acc_v, CH)

        def red_body(t, _):
            pltpu.sync_copy(shared.at[t, pl.ds(s * CH, CH)], tmp_v)

            def add_body(i, _):
                acc_v[pl.ds(i * L, L)] = (
                    acc_v[pl.ds(i * L, L)] + tmp_v[pl.ds(i * L, L)])
                return 0
            lax.fori_loop(0, CH // L, add_body, 0)
            return 0
        lax.fori_loop(0, NS, red_body, 0)

        pltpu.sync_copy(acc_v, out_hbm.at[c, pl.ds(s * CH, CH)])

    return deg_kernel


def _make_main_kernel():
    mesh = plsc.VectorSubcoreMesh(core_axis_name="c", subcore_axis_name="s")

    @functools.partial(
        pl.kernel,
        mesh=mesh,
        out_type=(
            jax.ShapeDtypeStruct((N_NODES, D_FEAT), _f32),
            jax.ShapeDtypeStruct((N_NODES, D_FEAT), _f32),
        ),
        scratch_types=[
            pltpu.VMEM((NPAD,), _f32),
            pltpu.VMEM((EPW,), _i32),
            pltpu.VMEM((EPW,), _i32),
            pltpu.VMEM((EPW,), _f32),
            pltpu.VMEM((LSZ,), _i32),
            pltpu.VMEM((LSZ,), _i32),
            pltpu.VMEM((LSZ,), _f32),
            pltpu.VMEM((K,), _i32),
            pltpu.VMEM((K,), _i32),
            pltpu.VMEM((K,), _i32),
            pltpu.VMEM((K,), _i32),
            pltpu.VMEM((K, D_FEAT), _f32),
            pltpu.VMEM((K, D_FEAT), _f32),
            pltpu.VMEM((CH,), _f32),
            pltpu.VMEM((CH,), _f32),
            pltpu.VMEM_SHARED((NPAD,), _f32),
            pltpu.VMEM_SHARED((SEG, D_FEAT), _f32),
            pltpu.SemaphoreType.DMA,
            pltpu.SemaphoreType.DMA,
        ],
        **_SC_PARAMS,
    )
    def main_kernel(x_hbm, src_hbm, dst_hbm, w_hbm, degp_hbm, out0_hbm,
                    out1_hbm, dis_v, src_v, dst_v, w_v, sl_v, dl_v, nl_v,
                    gi0_v, si0_v, gi1_v, si1_v, rows0_v, rows1_v, t0_v, t1_v,
                    shared_dis, acc, sem0, sem1):
        c = lax.axis_index("c")
        s = lax.axis_index("s")
        base = (c * NS + s) * EPW

        pltpu.sync_copy(degp_hbm.at[0, pl.ds(s * CH, CH)], t0_v)
        pltpu.sync_copy(degp_hbm.at[1, pl.ds(s * CH, CH)], t1_v)

        def dis_body(i, _):
            d = t0_v[pl.ds(i * L, L)] + t1_v[pl.ds(i * L, L)]
            y = _newton_rsqrt(jnp.maximum(d, 1e-12))
            t0_v[pl.ds(i * L, L)] = jnp.where(d > 0.0, y, 0.0)
            return 0
        lax.fori_loop(0, CH // L, dis_body, 0)
        pltpu.sync_copy(t0_v, shared_dis.at[pl.ds(s * CH, CH)])
        plsc.subcore_barrier()
        pltpu.sync_copy(shared_dis, dis_v)

        pltpu.sync_copy(src_hbm.at[pl.ds(base, EPW)], src_v)
        pltpu.sync_copy(dst_hbm.at[pl.ds(base, EPW)], dst_v)
        pltpu.sync_copy(w_hbm.at[pl.ds(base, EPW)], w_v)

        def norm_body(i, _):
            sv = src_v[pl.ds(i * L, L)]
            dv = dst_v[pl.ds(i * L, L)]
            w_v[pl.ds(i * L, L)] = (
                plsc.load_gather(dis_v, [sv]) * w_v[pl.ds(i * L, L)]
                * plsc.load_gather(dis_v, [dv]))
            return 0
        lax.fori_loop(0, EPW // L, norm_body, 0)

        def fill_idx(gi, si, b):
            for g in range(K // L):
                gi[pl.ds(g * L, L)] = sl_v[pl.ds(b * K + g * L, L)]
                si[pl.ds(g * L, L)] = dl_v[pl.ds(b * K + g * L, L)]

        def scale(rows, b):
            def scale_g(g, _):
                for jj in range(L):
                    j = g * L + jj
                    nsp = plsc.load_gather(
                        nl_v, [jnp.full((L,), b * K, _i32) + j])
                    for cb in range(D_FEAT // L):
                        rows[j, pl.ds(cb * L, L)] = (
                            rows[j, pl.ds(cb * L, L)] * nsp)
                return 0
            lax.fori_loop(0, K // L, scale_g, 0)

        for p in range(NPASS):
            def zrows_body(j, _):
                for cb in range(D_FEAT // L):
                    rows0_v[j, pl.ds(cb * L, L)] = jnp.zeros((L,), _f32)
                return 0
            lax.fori_loop(0, K, zrows_body, 0)
            for r in range(RPT // K):
                pltpu.sync_copy(rows0_v, acc.at[pl.ds(s * RPT + r * K, K)])
            ated = (RPT // K) * K
            if ated < RPT:
                pltpu.sync_copy(rows0_v.at[pl.ds(0, RPT - ated)],
                                acc.at[pl.ds(s * RPT + ated, RPT - ated)])
            plsc.subcore_barrier()

            def cmp_body(i, cnt):
                dv = dst_v[pl.ds(i * L, L)] - (p * SEG)
                msk = (dv >= 0) & (dv < SEG)
                plsc.store_compressed(sl_v.at[pl.ds(cnt, L)],
                                      src_v[pl.ds(i * L, L)], mask=msk)
                plsc.store_compressed(dl_v.at[pl.ds(cnt, L)], dv, mask=msk)
                plsc.store_compressed(nl_v.at[pl.ds(cnt, L)],
                                      w_v[pl.ds(i * L, L)], mask=msk)
                npop = jnp.max(plsc.all_reduce_population_count(msk))
                return cnt + npop
            cnt = lax.fori_loop(0, EPW // L, cmp_body, 0)

            for t in range(K // L):
                nl_v[pl.ds(cnt + t * L, L)] = jnp.zeros((L,), _f32)
                sl_v[pl.ds(cnt + t * L, L)] = jnp.zeros((L,), _i32)
                dl_v[pl.ds(cnt + t * L, L)] = jnp.zeros((L,), _i32)
            nb = (cnt + (K - 1)) // K

            @pl.when(nb > 0)
            def _():
                fill_idx(gi0_v, si0_v, 0)
                pltpu.async_copy(x_hbm.at[gi0_v], rows0_v, sem0)

            def pair_body(i, _):
                b0 = 2 * i
                b1 = b0 + 1

                @pl.when(b1 < nb)
                def _():
                    fill_idx(gi1_v, si1_v, b1)
                    pltpu.async_copy(x_hbm.at[gi1_v], rows1_v, sem1)

                pltpu.make_async_copy(x_hbm.at[gi0_v], rows0_v, sem0).wait()
                scale(rows0_v, b0)
                pltpu.sync_copy(rows0_v, acc.at[si0_v], add=True)

                @pl.when(b1 < nb)
                def _():
                    @pl.when(b0 + 2 < nb)
                    def _():
                        fill_idx(gi0_v, si0_v, b0 + 2)
                        pltpu.async_copy(x_hbm.at[gi0_v], rows0_v, sem0)

                    pltpu.make_async_copy(
                        x_hbm.at[gi1_v], rows1_v, sem1).wait()
                    scale(rows1_v, b1)
                    pltpu.sync_copy(rows1_v, acc.at[si1_v], add=True)
                return 0
            lax.fori_loop(0, (nb + 1) // 2, pair_body, 0)
            plsc.subcore_barrier()

            row0 = p * SEG + s * RPT
            lim = min(SEG, N_NODES - p * SEG)

            def wb(dst_ref):
                if lim == SEG:
                    pltpu.sync_copy(acc.at[pl.ds(s * RPT, RPT)],
                                    dst_ref.at[pl.ds(row0, RPT)])
                else:
                    @pl.when(s * RPT + RPT <= lim)
                    def _():
                        pltpu.sync_copy(acc.at[pl.ds(s * RPT, RPT)],
                                        dst_ref.at[pl.ds(row0, RPT)])

                    @pl.when((s * RPT < lim) & (s * RPT + RPT > lim))
                    def _():
                        part = lim % RPT
                        pltpu.sync_copy(acc.at[pl.ds(s * RPT, part)],
                                        dst_ref.at[pl.ds(row0, part)])

            @pl.when(c == 0)
            def _():
                wb(out0_hbm)

            @pl.when(c == 1)
            def _():
                wb(out1_hbm)

            plsc.subcore_barrier()

    return main_kernel


def _add_tc(a_ref, b_ref, o_ref):
    o_ref[...] = a_ref[...] + b_ref[...]


def kernel(x, edge_index, edge_weight):
    src = edge_index[0].astype(_i32)
    dst = edge_index[1].astype(_i32)
    w = edge_weight.astype(_f32)

    degp = _make_deg_kernel()(dst, w)
    p0, p1 = _make_main_kernel()(x, src, dst, w, degp)

    blk = 400
    out = pl.pallas_call(
        _add_tc,
        grid=(N_NODES // blk,),
        in_specs=[pl.BlockSpec((blk, D_FEAT), lambda i: (i, 0))] * 2,
        out_specs=pl.BlockSpec((blk, D_FEAT), lambda i: (i, 0)),
        out_shape=jax.ShapeDtypeStruct((N_NODES, D_FEAT), _f32),
    )(p0, p1)
    return out


# speedup vs baseline: 1.3474x; 1.0687x over previous
import functools

import jax
import jax.numpy as jnp
from jax import lax
from jax.experimental import pallas as pl
from jax.experimental.pallas import tpu as pltpu
from jax.experimental.pallas import tpu_sc as plsc

N_NODES = 10000
N_EDGES = 320000
D_FEAT = 128

NC = 2
NS = 16
NW = NC * NS
L = 16

EPW = N_EDGES // NW
NPAD = 10240
CH = NPAD // NS
NPASS = 2
SEG = 5120
RPT = SEG // NS
K = 32
LSZ = EPW + K + L
NBMAX = (EPW + K - 1) // K

_f32 = jnp.float32
_i32 = jnp.int32


def _zero_vmem(ref, n):
    def body(i, _):
        ref[pl.ds(i * L, L)] = jnp.zeros((L,), _f32)
        return 0
    lax.fori_loop(0, n // L, body, 0)


def _newton_rsqrt(d):
    i = plsc.bitcast(d, _i32)
    i = jnp.full((L,), 0x5F3759DF, _i32) - lax.shift_right_arithmetic(
        i, jnp.full((L,), 1, _i32))
    y = plsc.bitcast(i, _f32)
    half = d * 0.5
    for _ in range(3):
        y = y * (1.5 - half * y * y)
    return y


_SC_PARAMS = dict(
    compiler_params=pltpu.CompilerParams(
        needs_layout_passes=False, use_tc_tiling_on_sc=False),
)


def _make_deg_kernel():
    mesh = plsc.VectorSubcoreMesh(core_axis_name="c", subcore_axis_name="s")

    @functools.partial(
        pl.kernel,
        mesh=mesh,
        out_type=jax.ShapeDtypeStruct((NC, NPAD), _f32),
        scratch_types=[
            pltpu.VMEM((EPW,), _i32),
            pltpu.VMEM((EPW,), _f32),
            pltpu.VMEM((NPAD,), _f32),
            pltpu.VMEM_SHARED((NS, NPAD), _f32),
            pltpu.VMEM((CH,), _f32),
            pltpu.VMEM((CH,), _f32),
        ],
        **_SC_PARAMS,
    )
    def deg_kernel(dst_hbm, w_hbm, out_hbm, dst_v, w_v, deg_v, shared, tmp_v,
                   acc_v):
        c = lax.axis_index("c")
        s = lax.axis_index("s")
        base = (c * NS + s) * EPW
        pltpu.sync_copy(dst_hbm.at[pl.ds(base, EPW)], dst_v)
        pltpu.sync_copy(w_hbm.at[pl.ds(base, EPW)], w_v)
        _zero_vmem(deg_v, NPAD)

        def scatter_body(i, _):
            idx = dst_v[pl.ds(i * L, L)]
            wv = w_v[pl.ds(i * L, L)]
            plsc.addupdate_scatter(deg_v, [idx], wv)
            return 0
        lax.fori_loop(0, EPW // L, scatter_body, 0)

        pltpu.sync_copy(deg_v, shared.at[s])
        plsc.subcore_barrier()

        _zero_vmem(acc_v, CH)

        def red_body(t, _):
            pltpu.sync_copy(shared.at[t, pl.ds(s * CH, CH)], tmp_v)

            def add_body(i, _):
                acc_v[pl.ds(i * L, L)] = (
                    acc_v[pl.ds(i * L, L)] + tmp_v[pl.ds(i * L, L)])
                return 0
            lax.fori_loop(0, CH // L, add_body, 0)
            return 0
        lax.fori_loop(0, NS, red_body, 0)

        pltpu.sync_copy(acc_v, out_hbm.at[c, pl.ds(s * CH, CH)])

    return deg_kernel


def _make_main_kernel():
    mesh = plsc.VectorSubcoreMesh(core_axis_name="c", subcore_axis_name="s")

    @functools.partial(
        pl.kernel,
        mesh=mesh,
        out_type=(
            jax.ShapeDtypeStruct((N_NODES, D_FEAT), _f32),
            jax.ShapeDtypeStruct((N_NODES, D_FEAT), _f32),
        ),
        scratch_types=[
            pltpu.VMEM((NPAD,), _f32),
            pltpu.VMEM((EPW,), _i32),
            pltpu.VMEM((EPW,), _i32),
            pltpu.VMEM((EPW,), _f32),
            pltpu.VMEM((LSZ,), _i32),
            pltpu.VMEM((LSZ,), _i32),
            pltpu.VMEM((LSZ,), _f32),
            pltpu.VMEM((K,), _i32),
            pltpu.VMEM((K,), _i32),
            pltpu.VMEM((K,), _i32),
            pltpu.VMEM((K,), _i32),
            pltpu.VMEM((K, D_FEAT), _f32),
            pltpu.VMEM((K, D_FEAT), _f32),
            pltpu.VMEM((CH,), _f32),
            pltpu.VMEM((CH,), _f32),
            pltpu.VMEM_SHARED((NPAD,), _f32),
            pltpu.VMEM_SHARED((SEG, D_FEAT), _f32),
            pltpu.SemaphoreType.DMA,
            pltpu.SemaphoreType.DMA,
        ],
        **_SC_PARAMS,
    )
    def main_kernel(x_hbm, src_hbm, dst_hbm, w_hbm, degp_hbm, out0_hbm,
                    out1_hbm, dis_v, src_v, dst_v, w_v, sl_v, dl_v, nl_v,
                    gi0_v, si0_v, gi1_v, si1_v, rows0_v, rows1_v, t0_v, t1_v,
                    shared_dis, acc, sem0, sem1):
        c = lax.axis_index("c")
        s = lax.axis_index("s")
        base = (c * NS + s) * EPW

        pltpu.sync_copy(degp_hbm.at[0, pl.ds(s * CH, CH)], t0_v)
        pltpu.sync_copy(degp_hbm.at[1, pl.ds(s * CH, CH)], t1_v)

        def dis_body(i, _):
            d = t0_v[pl.ds(i * L, L)] + t1_v[pl.ds(i * L, L)]
            y = _newton_rsqrt(jnp.maximum(d, 1e-12))
            t0_v[pl.ds(i * L, L)] = jnp.where(d > 0.0, y, 0.0)
            return 0
        lax.fori_loop(0, CH // L, dis_body, 0)
        pltpu.sync_copy(t0_v, shared_dis.at[pl.ds(s * CH, CH)])
        plsc.subcore_barrier()
        pltpu.sync_copy(shared_dis, dis_v)

        pltpu.sync_copy(src_hbm.at[pl.ds(base, EPW)], src_v)
        pltpu.sync_copy(dst_hbm.at[pl.ds(base, EPW)], dst_v)
        pltpu.sync_copy(w_hbm.at[pl.ds(base, EPW)], w_v)

        def norm_body(i, _):
            sv = src_v[pl.ds(i * L, L)]
            dv = dst_v[pl.ds(i * L, L)]
            w_v[pl.ds(i * L, L)] = (
                plsc.load_gather(dis_v, [sv]) * w_v[pl.ds(i * L, L)]
                * plsc.load_gather(dis_v, [dv]))
            return 0
        lax.fori_loop(0, EPW // L, norm_body, 0)

        def fill_idx(gi, si, b):
            for g in range(K // L):
                gi[pl.ds(g * L, L)] = sl_v[pl.ds(b * K + g * L, L)]
                si[pl.ds(g * L, L)] = dl_v[pl.ds(b * K + g * L, L)]

        def scale(rows, b):
            def scale_g(g, _):
                for jj in range(L):
                    j = g * L + jj
                    nsp = plsc.load_gather(
                        nl_v, [jnp.full((L,), b * K, _i32) + j])
                    for cb in range(D_FEAT // L):
                        rows[j, pl.ds(cb * L, L)] = (
                            rows[j, pl.ds(cb * L, L)] * nsp)
                return 0
            lax.fori_loop(0, K // L, scale_g, 0)

        for p in range(NPASS):
            def zrows_body(j, _):
                for cb in range(D_FEAT // L):
                    rows0_v[j, pl.ds(cb * L, L)] = jnp.zeros((L,), _f32)
                return 0
            lax.fori_loop(0, K, zrows_body, 0)
            for r in range(RPT // K):
                pltpu.sync_copy(rows0_v, acc.at[pl.ds(s * RPT + r * K, K)])
            ated = (RPT // K) * K
            if ated < RPT:
                pltpu.sync_copy(rows0_v.at[pl.ds(0, RPT - ated)],
                                acc.at[pl.ds(s * RPT + ated, RPT - ated)])
            plsc.subcore_barrier()

            def cmp_body(i, cnt):
                dv = dst_v[pl.ds(i * L, L)] - (p * SEG)
                msk = (dv >= 0) & (dv < SEG)
                plsc.store_compressed(sl_v.at[pl.ds(cnt, L)],
                                      src_v[pl.ds(i * L, L)], mask=msk)
                plsc.store_compressed(dl_v.at[pl.ds(cnt, L)], dv, mask=msk)
                plsc.store_compressed(nl_v.at[pl.ds(cnt, L)],
                                      w_v[pl.ds(i * L, L)], mask=msk)
                npop = jnp.max(plsc.all_reduce_population_count(msk))
                return cnt + npop
            cnt = lax.fori_loop(0, EPW // L, cmp_body, 0)

            for t in range(K // L):
                nl_v[pl.ds(cnt + t * L, L)] = jnp.zeros((L,), _f32)
                sl_v[pl.ds(cnt + t * L, L)] = jnp.zeros((L,), _i32)
                dl_v[pl.ds(cnt + t * L, L)] = jnp.zeros((L,), _i32)
            nb = (cnt + (K - 1)) // K

            @pl.when(nb > 0)
            def _():
                fill_idx(gi0_v, si0_v, 0)
                pltpu.async_copy(x_hbm.at[gi0_v], rows0_v, sem0)

            def pair_body(i, _):
                b0 = 2 * i
                b1 = b0 + 1

                @pl.when(b1 < nb)
                def _():
                    fill_idx(gi1_v, si1_v, b1)
                    pltpu.async_copy(x_hbm.at[gi1_v], rows1_v, sem1)

                pltpu.make_async_copy(x_hbm.at[gi0_v], rows0_v, sem0).wait()
                scale(rows0_v, b0)
                pltpu.sync_copy(rows0_v, acc.at[si0_v], add=True)

                @pl.when(b1 < nb)
                def _():
                    @pl.when(b0 + 2 < nb)
                    def _():
                        fill_idx(gi0_v, si0_v, b0 + 2)
                        pltpu.async_copy(x_hbm.at[gi0_v], rows0_v, sem0)

                    pltpu.make_async_copy(
                        x_hbm.at[gi1_v], rows1_v, sem1).wait()
                    scale(rows1_v, b1)
                    pltpu.sync_copy(rows1_v, acc.at[si1_v], add=True)
                return 0
            lax.fori_loop(0, (nb + 1) // 2, pair_body, 0)
            plsc.subcore_barrier()

            row0 = p * SEG + s * RPT
            lim = min(SEG, N_NODES - p * SEG)

            def wb(dst_ref):
                if lim == SEG:
                    pltpu.sync_copy(acc.at[pl.ds(s * RPT, RPT)],
                                    dst_ref.at[pl.ds(row0, RPT)])
                else:
                    @pl.when(s * RPT + RPT <= lim)
                    def _():
                        pltpu.sync_copy(acc.at[pl.ds(s * RPT, RPT)],
                                        dst_ref.at[pl.ds(row0, RPT)])

                    @pl.when((s * RPT < lim) & (s * RPT + RPT > lim))
                    def _():
                        part = lim % RPT
                        pltpu.sync_copy(acc.at[pl.ds(s * RPT, part)],
                                        dst_ref.at[pl.ds(row0, part)])

            @pl.when(c == 0)
            def _():
                wb(out0_hbm)

            @pl.when(c == 1)
            def _():
                wb(out1_hbm)

            plsc.subcore_barrier()

    return main_kernel


def _add_tc(a_ref, b_ref, o_ref):
    o_ref[...] = a_ref[...] + b_ref[...]


def kernel(x, edge_index, edge_weight):
    src = edge_index[0].astype(_i32)
    dst = edge_index[1].astype(_i32)
    w = edge_weight.astype(_f32)

    degp = _make_deg_kernel()(dst, w)
    p0, p1 = _make_main_kernel()(x, src, dst, w, degp)

    blk = 400
    out = pl.pallas_call(
        _add_tc,
        grid=(N_NODES // blk,),
        in_specs=[pl.BlockSpec((blk, D_FEAT), lambda i: (i, 0))] * 2,
        out_specs=pl.BlockSpec((blk, D_FEAT), lambda i: (i, 0)),
        out_shape=jax.ShapeDtypeStruct((N_NODES, D_FEAT), _f32),
    )(p0, p1)
    return out


# speedup vs baseline: 1.4876x; 1.1041x over previous
import functools

import jax
import jax.numpy as jnp
from jax import lax
from jax.experimental import pallas as pl
from jax.experimental.pallas import tpu as pltpu
from jax.experimental.pallas import tpu_sc as plsc

N_NODES = 10000
N_EDGES = 320000
D_FEAT = 128

NC = 2
NS = 16
NW = NC * NS
L = 16

EPW = N_EDGES // NW
NPAD = 10240
CH = NPAD // NS
NPASS = 2
SEG = 5120
RPT = SEG // NS
K = 32
LSZ = EPW + K + L
NBMAX = (EPW + K - 1) // K

_f32 = jnp.float32
_i32 = jnp.int32


def _zero_vmem(ref, n):
    def body(i, _):
        ref[pl.ds(i * L, L)] = jnp.zeros((L,), _f32)
        return 0
    lax.fori_loop(0, n // L, body, 0)


def _newton_rsqrt(d):
    i = plsc.bitcast(d, _i32)
    i = jnp.full((L,), 0x5F3759DF, _i32) - lax.shift_right_arithmetic(
        i, jnp.full((L,), 1, _i32))
    y = plsc.bitcast(i, _f32)
    half = d * 0.5
    for _ in range(3):
        y = y * (1.5 - half * y * y)
    return y


_SC_PARAMS = dict(
    compiler_params=pltpu.CompilerParams(
        needs_layout_passes=False, use_tc_tiling_on_sc=False),
)


def _make_deg_kernel():
    mesh = plsc.VectorSubcoreMesh(core_axis_name="c", subcore_axis_name="s")

    @functools.partial(
        pl.kernel,
        mesh=mesh,
        out_type=jax.ShapeDtypeStruct((NC, NPAD), _f32),
        scratch_types=[
            pltpu.VMEM((EPW,), _i32),
            pltpu.VMEM((EPW,), _f32),
            pltpu.VMEM((NPAD,), _f32),
            pltpu.VMEM_SHARED((NS, NPAD), _f32),
            pltpu.VMEM((CH,), _f32),
            pltpu.VMEM((CH,), _f32),
        ],
        **_SC_PARAMS,
    )
    def deg_kernel(dst_hbm, w_hbm, out_hbm, dst_v, w_v, deg_v, shared, tmp_v,
                   acc_v):
        c = lax.axis_index("c")
        s = lax.axis_index("s")
        base = (c * NS + s) * EPW
        pltpu.sync_copy(dst_hbm.at[pl.ds(base, EPW)], dst_v)
        pltpu.sync_copy(w_hbm.at[pl.ds(base, EPW)], w_v)
        _zero_vmem(deg_v, NPAD)

        def scatter_body(i, _):
            idx = dst_v[pl.ds(i * L, L)]
            wv = w_v[pl.ds(i * L, L)]
            plsc.addupdate_scatter(deg_v, [idx], wv)
            return 0
        lax.fori_loop(0, EPW // L, scatter_body, 0)

        pltpu.sync_copy(deg_v, shared.at[s])
        plsc.subcore_barrier()

        _zero_vmem(acc_v, CH)

        def red_body(t, _):
            pltpu.sync_copy(shared.at[t, pl.ds(s * CH, CH)], tmp_v)

            def add_body(i, _):
                acc_v[pl.ds(i * L, L)] = (
                    acc_v[pl.ds(i * L, L)] + tmp_v[pl.ds(i * L, L)])
                return 0
            lax.fori_loop(0, CH // L, add_body, 0)
            return 0
        lax.fori_loop(0, NS, red_body, 0)

        pltpu.sync_copy(acc_v, out_hbm.at[c, pl.ds(s * CH, CH)])

    return deg_kernel


def _make_main_kernel():
    mesh = plsc.VectorSubcoreMesh(core_axis_name="c", subcore_axis_name="s")

    @functools.partial(
        pl.kernel,
        mesh=mesh,
        out_type=(
            jax.ShapeDtypeStruct((N_NODES, D_FEAT), _f32),
            jax.ShapeDtypeStruct((N_NODES, D_FEAT), _f32),
        ),
        scratch_types=[
            pltpu.VMEM((NPAD,), _f32),
            pltpu.VMEM((EPW,), _i32),
            pltpu.VMEM((EPW,), _i32),
            pltpu.VMEM((EPW,), _f32),
            pltpu.VMEM((LSZ,), _i32),
            pltpu.VMEM((LSZ,), _i32),
            pltpu.VMEM((LSZ,), _f32),
            pltpu.VMEM((K,), _i32),
            pltpu.VMEM((K,), _i32),
            pltpu.VMEM((K,), _i32),
            pltpu.VMEM((K,), _i32),
            pltpu.VMEM((K,), _i32),
            pltpu.VMEM((K,), _i32),
            pltpu.VMEM((K, D_FEAT), _f32),
            pltpu.VMEM((K, D_FEAT), _f32),
            pltpu.VMEM((K, D_FEAT), _f32),
            pltpu.VMEM((CH,), _f32),
            pltpu.VMEM((CH,), _f32),
            pltpu.VMEM_SHARED((NPAD,), _f32),
            pltpu.VMEM_SHARED((SEG, D_FEAT), _f32),
            pltpu.SemaphoreType.DMA,
            pltpu.SemaphoreType.DMA,
            pltpu.SemaphoreType.DMA,
        ],
        **_SC_PARAMS,
    )
    def main_kernel(x_hbm, src_hbm, dst_hbm, w_hbm, degp_hbm, out0_hbm,
                    out1_hbm, dis_v, src_v, dst_v, w_v, sl_v, dl_v, nl_v,
                    gi0_v, si0_v, gi1_v, si1_v, gi2_v, si2_v, rows0_v,
                    rows1_v, rows2_v, t0_v, t1_v, shared_dis, acc,
                    sem0, sem1, sem2):
        c = lax.axis_index("c")
        s = lax.axis_index("s")
        base = (c * NS + s) * EPW

        pltpu.sync_copy(degp_hbm.at[0, pl.ds(s * CH, CH)], t0_v)
        pltpu.sync_copy(degp_hbm.at[1, pl.ds(s * CH, CH)], t1_v)

        def dis_body(i, _):
            d = t0_v[pl.ds(i * L, L)] + t1_v[pl.ds(i * L, L)]
            y = _newton_rsqrt(jnp.maximum(d, 1e-12))
            t0_v[pl.ds(i * L, L)] = jnp.where(d > 0.0, y, 0.0)
            return 0
        lax.fori_loop(0, CH // L, dis_body, 0)
        pltpu.sync_copy(t0_v, shared_dis.at[pl.ds(s * CH, CH)])
        plsc.subcore_barrier()
        pltpu.sync_copy(shared_dis, dis_v)

        pltpu.sync_copy(src_hbm.at[pl.ds(base, EPW)], src_v)
        pltpu.sync_copy(dst_hbm.at[pl.ds(base, EPW)], dst_v)
        pltpu.sync_copy(w_hbm.at[pl.ds(base, EPW)], w_v)

        def norm_body(i, _):
            sv = src_v[pl.ds(i * L, L)]
            dv = dst_v[pl.ds(i * L, L)]
            w_v[pl.ds(i * L, L)] = (
                plsc.load_gather(dis_v, [sv]) * w_v[pl.ds(i * L, L)]
                * plsc.load_gather(dis_v, [dv]))
            return 0
        lax.fori_loop(0, EPW // L, norm_body, 0)

        def fill_idx(gi, si, b):
            for g in range(K // L):
                gi[pl.ds(g * L, L)] = sl_v[pl.ds(b * K + g * L, L)]
                si[pl.ds(g * L, L)] = dl_v[pl.ds(b * K + g * L, L)]

        def scale(rows, b):
            def scale_g(g, _):
                for jj in range(L):
                    j = g * L + jj
                    nsp = plsc.load_gather(
                        nl_v, [jnp.full((L,), b * K, _i32) + j])
                    for cb in range(D_FEAT // L):
                        rows[j, pl.ds(cb * L, L)] = (
                            rows[j, pl.ds(cb * L, L)] * nsp)
                return 0
            lax.fori_loop(0, K // L, scale_g, 0)

        for p in range(NPASS):
            def zrows_body(j, _):
                for cb in range(D_FEAT // L):
                    rows0_v[j, pl.ds(cb * L, L)] = jnp.zeros((L,), _f32)
                return 0
            lax.fori_loop(0, K, zrows_body, 0)
            for r in range(RPT // K):
                pltpu.sync_copy(rows0_v, acc.at[pl.ds(s * RPT + r * K, K)])
            ated = (RPT // K) * K
            if ated < RPT:
                pltpu.sync_copy(rows0_v.at[pl.ds(0, RPT - ated)],
                                acc.at[pl.ds(s * RPT + ated, RPT - ated)])
            plsc.subcore_barrier()

            def cmp_body(i, cnt):
                dv = dst_v[pl.ds(i * L, L)] - (p * SEG)
                msk = (dv >= 0) & (dv < SEG)
                plsc.store_compressed(sl_v.at[pl.ds(cnt, L)],
                                      src_v[pl.ds(i * L, L)], mask=msk)
                plsc.store_compressed(dl_v.at[pl.ds(cnt, L)], dv, mask=msk)
                plsc.store_compressed(nl_v.at[pl.ds(cnt, L)],
                                      w_v[pl.ds(i * L, L)], mask=msk)
                npop = jnp.max(plsc.all_reduce_population_count(msk))
                return cnt + npop
            cnt = lax.fori_loop(0, EPW // L, cmp_body, 0)

            for t in range(K // L):
                nl_v[pl.ds(cnt + t * L, L)] = jnp.zeros((L,), _f32)
                sl_v[pl.ds(cnt + t * L, L)] = jnp.zeros((L,), _i32)
                dl_v[pl.ds(cnt + t * L, L)] = jnp.zeros((L,), _i32)
            nb = (cnt + (K - 1)) // K

            bufs = ((gi0_v, si0_v, rows0_v, sem0),
                    (gi1_v, si1_v, rows1_v, sem1),
                    (gi2_v, si2_v, rows2_v, sem2))
            for k in range(2):
                @pl.when(k < nb)
                def _(k=k):
                    gi, si, rows, sem = bufs[k]
                    fill_idx(gi, si, k)
                    pltpu.async_copy(x_hbm.at[gi], rows, sem)

            def tri_body(i, _):
                for bb in range(3):
                    b = 3 * i + bb

                    @pl.when(b < nb)
                    def _(b=b, bb=bb):
                        gin, sin, rowsn, semn = bufs[(bb + 2) % 3]

                        @pl.when(b + 2 < nb)
                        def _():
                            fill_idx(gin, sin, b + 2)
                            pltpu.async_copy(x_hbm.at[gin], rowsn, semn)

                        gi, si, rows, sem = bufs[bb]
                        pltpu.make_async_copy(x_hbm.at[gi], rows, sem).wait()
                        scale(rows, b)
                        pltpu.sync_copy(rows, acc.at[si], add=True)
                return 0
            lax.fori_loop(0, (nb + 2) // 3, tri_body, 0)
            plsc.subcore_barrier()

            row0 = p * SEG + s * RPT
            lim = min(SEG, N_NODES - p * SEG)

            def wb(dst_ref):
                if lim == SEG:
                    pltpu.sync_copy(acc.at[pl.ds(s * RPT, RPT)],
                                    dst_ref.at[pl.ds(row0, RPT)])
                else:
                    @pl.when(s * RPT + RPT <= lim)
                    def _():
                        pltpu.sync_copy(acc.at[pl.ds(s * RPT, RPT)],
                                        dst_ref.at[pl.ds(row0, RPT)])

                    @pl.when((s * RPT < lim) & (s * RPT + RPT > lim))
                    def _():
                        part = lim % RPT
                        pltpu.sync_copy(acc.at[pl.ds(s * RPT, part)],
                                        dst_ref.at[pl.ds(row0, part)])

            @pl.when(c == 0)
            def _():
                wb(out0_hbm)

            @pl.when(c == 1)
            def _():
                wb(out1_hbm)

            plsc.subcore_barrier()

    return main_kernel


def _add_tc(a_ref, b_ref, o_ref):
    o_ref[...] = a_ref[...] + b_ref[...]


def kernel(x, edge_index, edge_weight):
    src = edge_index[0].astype(_i32)
    dst = edge_index[1].astype(_i32)
    w = edge_weight.astype(_f32)

    degp = _make_deg_kernel()(dst, w)
    p0, p1 = _make_main_kernel()(x, src, dst, w, degp)

    blk = 400
    out = pl.pallas_call(
        _add_tc,
        grid=(N_NODES // blk,),
        in_specs=[pl.BlockSpec((blk, D_FEAT), lambda i: (i, 0))] * 2,
        out_specs=pl.BlockSpec((blk, D_FEAT), lambda i: (i, 0)),
        out_shape=jax.ShapeDtypeStruct((N_NODES, D_FEAT), _f32),
    )(p0, p1)
    return out
